# Initial kernel scaffold; baseline (speedup 1.0000x reference)
import jax
import jax.numpy as jnp
from jax.experimental import pallas as pl


def kernel(xyz, feat):
    raise NotImplementedError("write your pallas kernel here")


# speedup vs baseline: 11.2864x; 11.2864x over previous
import functools

import jax
import jax.numpy as jnp
from jax import lax
from jax.experimental import pallas as pl
from jax.experimental.pallas import tpu as pltpu
from jax.experimental.pallas import tpu_sc as plsc

KNB = 32
QB = 256
NC = 2
NS = 16
NW = NC * NS
CH = 256
XW = 16


def _knn_body(xyzq_ref, xyzt_ref, idx_ref):
    b = pl.program_id(0)
    n = xyzt_ref.shape[-1]
    lhs = xyzq_ref[0]
    rhs = xyzt_ref[0]
    qx, qy, qz = lhs[:, 0:1], lhs[:, 1:2], lhs[:, 2:3]
    q2 = (qx * qx + qy * qy) + qz * qz
    rx, ry, rz = rhs[0:1, :], rhs[1:2, :], rhs[2:3, :]
    r2 = (rx * rx + ry * ry) + rz * rz
    inner = jnp.dot(lhs, rhs, preferred_element_type=jnp.float32)
    d2 = (q2 + r2) - 2.0 * inner
    lanes = lax.broadcasted_iota(jnp.int32, d2.shape, 1)
    off = b * n
    cols = []
    for _ in range(KNB):
        m = jnp.min(d2, axis=1, keepdims=True)
        am = jnp.min(jnp.where(d2 == m, lanes, n), axis=1, keepdims=True)
        cols.append(am + off)
        d2 = jnp.where(lanes == am, jnp.float32(jnp.inf), d2)
    idx_ref[0] = jnp.concatenate(cols, axis=1)


def _tc_knn(xyzp8, xyzt8):
    b, n, _ = xyzp8.shape
    grid = (b, n // QB)
    return pl.pallas_call(
        _knn_body,
        grid=grid,
        in_specs=[
            pl.BlockSpec((1, QB, 8), lambda i, j: (i, j, 0)),
            pl.BlockSpec((1, 8, n), lambda i, j: (i, 0, 0)),
        ],
        out_specs=pl.BlockSpec((1, QB, KNB), lambda i, j: (i, j, 0)),
        out_shape=jax.ShapeDtypeStruct((b, n, KNB), jnp.int32),
        compiler_params=pltpu.CompilerParams(
            dimension_semantics=("parallel", "parallel"),
        ),
    )(xyzp8, xyzt8)


def _sc_gather(featf, xyzw, idxf):
    r_total = idxf.shape[0] * idxf.shape[1]
    c = featf.shape[1]
    npts = xyzw.shape[0] // 4
    rpw = r_total // NW
    nch = rpw // CH
    mesh = plsc.VectorSubcoreMesh(core_axis_name="c", subcore_axis_name="s")

    @functools.partial(
        pl.kernel,
        mesh=mesh,
        out_type=(
            jax.ShapeDtypeStruct((r_total, c), jnp.float32),
            jax.ShapeDtypeStruct((r_total * 3,), jnp.float32),
        ),
        scratch_types=[
            pltpu.VMEM((rpw // 128, 128), jnp.int32),
            pltpu.VMEM((CH, c), jnp.float32),
            pltpu.VMEM((npts * 4,), jnp.float32),
            pltpu.VMEM((CH * 3,), jnp.float32),
            pltpu.SemaphoreType.DMA,
        ],
        compiler_params=pltpu.CompilerParams(needs_layout_passes=False),
    )
    def gather(feat_hbm, xyzw_hbm, idx_hbm, featg_hbm, xyzg_hbm,
               idx_v, featr_v, xyztab_v, xyzo_v, sem):
        wid = lax.axis_index("s") * NC + lax.axis_index("c")
        base0 = wid * rpw
        pltpu.sync_copy(idx_hbm.at[pl.ds(wid * (rpw // 128), rpw // 128)],
                        idx_v)
        pltpu.sync_copy(xyzw_hbm, xyztab_v)
        iota16 = lax.iota(jnp.int32, 16)
        qg0 = wid * (rpw // KNB)

        def chunk(ci, carry):
            base = base0 + ci * CH
            cps = []
            for j in range(CH // 128):
                cps.append(pltpu.async_copy(
                    feat_hbm.at[idx_v.at[ci * (CH // 128) + j]],
                    featr_v.at[pl.ds(j * 128, 128)], sem))
            for v in range(CH * 3 // 16):
                wl = ci * (CH * 3) + v * 16 + iota16
                r = wl // 3
                col = wl - r * 3
                q = qg0 + (r >> 5)
                nidx = plsc.load_gather(idx_v, [r >> 7, r & 127])
                xv = plsc.load_gather(xyztab_v, [nidx * 4 + col])
                cv = plsc.load_gather(xyztab_v, [q * 4 + col])
                xyzo_v[pl.ds(v * 16, 16)] = xv - cv
            for cp in cps:
                cp.wait()
            pltpu.sync_copy(featr_v, featg_hbm.at[pl.ds(base, CH)])
            pltpu.sync_copy(xyzo_v, xyzg_hbm.at[pl.ds(base * 3, CH * 3)])
            return carry

        lax.fori_loop(0, nch, chunk, 0)

    return gather(featf, xyzw, idxf)


def kernel(xyz, feat):
    b, n, _ = xyz.shape
    c = feat.shape[-1]
    xyzp8 = jnp.pad(xyz, ((0, 0), (0, 0), (0, 5)))
    xyzt8 = jnp.transpose(xyzp8, (0, 2, 1))
    idx = _tc_knn(xyzp8, xyzt8)
    featf = feat.reshape(b * n, c)
    xyzw = jnp.pad(xyz.reshape(b * n, 3), ((0, 0), (0, 1))).reshape(-1)
    featg, xyzg = _sc_gather(featf, xyzw, idx.reshape(-1, 128))
    neighborhood = xyzg.reshape(b, n, KNB, 3)
    neighborhood_feat = featg.reshape(b, n, KNB, c)
    return (neighborhood, neighborhood_feat)


# speedup vs baseline: 13.7331x; 1.2168x over previous
import functools

import jax
import jax.numpy as jnp
from jax import lax
from jax.experimental import pallas as pl
from jax.experimental.pallas import tpu as pltpu
from jax.experimental.pallas import tpu_sc as plsc

KNB = 32
QB = 256
NC = 2
NS = 16
NW = NC * NS
CH = 256
XW = 16


def _knn_body(xyzq_ref, xyzt_ref, idx_ref):
    b = pl.program_id(0)
    n = xyzt_ref.shape[-1]
    lhs = xyzq_ref[0]
    rhs = xyzt_ref[0]
    qx, qy, qz = lhs[:, 0:1], lhs[:, 1:2], lhs[:, 2:3]
    q2 = (qx * qx + qy * qy) + qz * qz
    rx, ry, rz = rhs[0:1, :], rhs[1:2, :], rhs[2:3, :]
    r2 = (rx * rx + ry * ry) + rz * rz
    inner = jnp.dot(lhs, rhs, preferred_element_type=jnp.float32)
    d2 = (q2 + r2) - 2.0 * inner
    lanes = lax.broadcasted_iota(jnp.int32, d2.shape, 1).astype(jnp.float32)
    nf = jnp.float32(n)
    cols = []
    for _ in range(KNB):
        m = jnp.min(d2, axis=1, keepdims=True)
        am = jnp.min(jnp.where(d2 == m, lanes, nf), axis=1, keepdims=True)
        cols.append(am)
        d2 = jnp.where(lanes == am, jnp.float32(jnp.inf), d2)
    off = b * n
    idx_ref[0] = jnp.concatenate(cols, axis=1).astype(jnp.int32) + off


def _tc_knn(xyzp8, xyzt8):
    b, n, _ = xyzp8.shape
    grid = (b, n // QB)
    return pl.pallas_call(
        _knn_body,
        grid=grid,
        in_specs=[
            pl.BlockSpec((1, QB, 8), lambda i, j: (i, j, 0)),
            pl.BlockSpec((1, 8, n), lambda i, j: (i, 0, 0)),
        ],
        out_specs=pl.BlockSpec((1, QB, KNB), lambda i, j: (i, j, 0)),
        out_shape=jax.ShapeDtypeStruct((b, n, KNB), jnp.int32),
        compiler_params=pltpu.CompilerParams(
            dimension_semantics=("parallel", "parallel"),
        ),
    )(xyzp8, xyzt8)


def _sc_gather(featf, xyzw, idxf):
    r_total = idxf.shape[0] * idxf.shape[1]
    c = featf.shape[1]
    npts = xyzw.shape[0] // 4
    rpw = r_total // NW
    nch = rpw // CH
    mesh = plsc.VectorSubcoreMesh(core_axis_name="c", subcore_axis_name="s")

    @functools.partial(
        pl.kernel,
        mesh=mesh,
        out_type=(
            jax.ShapeDtypeStruct((r_total, c), jnp.float32),
            jax.ShapeDtypeStruct((r_total * 3,), jnp.float32),
        ),
        scratch_types=[
            pltpu.VMEM((rpw // 128, 128), jnp.int32),
            pltpu.VMEM((CH, c), jnp.float32),
            pltpu.VMEM((npts * 4,), jnp.float32),
            pltpu.VMEM((CH * 3,), jnp.float32),
            pltpu.SemaphoreType.DMA,
        ],
        compiler_params=pltpu.CompilerParams(needs_layout_passes=False),
    )
    def gather(feat_hbm, xyzw_hbm, idx_hbm, featg_hbm, xyzg_hbm,
               idx_v, featr_v, xyztab_v, xyzo_v, sem):
        wid = lax.axis_index("s") * NC + lax.axis_index("c")
        base0 = wid * rpw
        pltpu.sync_copy(idx_hbm.at[pl.ds(wid * (rpw // 128), rpw // 128)],
                        idx_v)
        pltpu.sync_copy(xyzw_hbm, xyztab_v)
        iota16 = lax.iota(jnp.int32, 16)
        qg0 = wid * (rpw // KNB)

        def chunk(ci, carry):
            base = base0 + ci * CH
            cps = []
            for j in range(CH // 128):
                cps.append(pltpu.async_copy(
                    feat_hbm.at[idx_v.at[ci * (CH // 128) + j]],
                    featr_v.at[pl.ds(j * 128, 128)], sem))
            for v in range(CH * 3 // 16):
                wl = ci * (CH * 3) + v * 16 + iota16
                r = wl // 3
                col = wl - r * 3
                q = qg0 + (r >> 5)
                nidx = plsc.load_gather(idx_v, [r >> 7, r & 127])
                xv = plsc.load_gather(xyztab_v, [nidx * 4 + col])
                cv = plsc.load_gather(xyztab_v, [q * 4 + col])
                xyzo_v[pl.ds(v * 16, 16)] = xv - cv
            for cp in cps:
                cp.wait()
            pltpu.sync_copy(featr_v, featg_hbm.at[pl.ds(base, CH)])
            pltpu.sync_copy(xyzo_v, xyzg_hbm.at[pl.ds(base * 3, CH * 3)])
            return carry

        lax.fori_loop(0, nch, chunk, 0)

    return gather(featf, xyzw, idxf)


def kernel(xyz, feat):
    b, n, _ = xyz.shape
    c = feat.shape[-1]
    xyzp8 = jnp.pad(xyz, ((0, 0), (0, 0), (0, 5)))
    xyzt8 = jnp.transpose(xyzp8, (0, 2, 1))
    idx = _tc_knn(xyzp8, xyzt8)
    featf = feat.reshape(b * n, c)
    xyzw = jnp.pad(xyz.reshape(b * n, 3), ((0, 0), (0, 1))).reshape(-1)
    featg, xyzg = _sc_gather(featf, xyzw, idx.reshape(-1, 128))
    neighborhood = xyzg.reshape(b, n, KNB, 3)
    neighborhood_feat = featg.reshape(b, n, KNB, c)
    return (neighborhood, neighborhood_feat)


# speedup vs baseline: 14.0418x; 1.0225x over previous
import functools

import jax
import jax.numpy as jnp
from jax import lax
from jax.experimental import pallas as pl
from jax.experimental.pallas import tpu as pltpu
from jax.experimental.pallas import tpu_sc as plsc

KNB = 32
QB = 256
NC = 2
NS = 16
NW = NC * NS
CH = 256
XW = 16


def _knn_body(xyzq_ref, xyzt_ref, idx_ref):
    b = pl.program_id(0)
    n = xyzt_ref.shape[-1]
    lhs = xyzq_ref[0]
    rhs = xyzt_ref[0]
    qx, qy, qz = lhs[:, 0:1], lhs[:, 1:2], lhs[:, 2:3]
    q2 = (qx * qx + qy * qy) + qz * qz
    rx, ry, rz = rhs[0:1, :], rhs[1:2, :], rhs[2:3, :]
    r2 = (rx * rx + ry * ry) + rz * rz
    inner = jnp.dot(lhs, rhs, preferred_element_type=jnp.float32)
    d2 = (q2 + r2) - 2.0 * inner
    lanes = lax.broadcasted_iota(jnp.int32, d2.shape, 1).astype(jnp.float32)
    nf = jnp.float32(n)
    cols = []
    for _ in range(KNB):
        m = jnp.min(d2, axis=1, keepdims=True)
        am = jnp.min(jnp.where(d2 == m, lanes, nf), axis=1, keepdims=True)
        cols.append(am)
        d2 = jnp.where(lanes == am, jnp.float32(jnp.inf), d2)
    off = b * n
    idx_ref[0] = jnp.concatenate(cols, axis=1).astype(jnp.int32) + off


def _tc_knn(xyzp8, xyzt8):
    b, n, _ = xyzp8.shape
    grid = (b, n // QB)
    return pl.pallas_call(
        _knn_body,
        grid=grid,
        in_specs=[
            pl.BlockSpec((1, QB, 8), lambda i, j: (i, j, 0)),
            pl.BlockSpec((1, 8, n), lambda i, j: (i, 0, 0)),
        ],
        out_specs=pl.BlockSpec((1, QB, KNB), lambda i, j: (i, j, 0)),
        out_shape=jax.ShapeDtypeStruct((b, n, KNB), jnp.int32),
        compiler_params=pltpu.CompilerParams(
            dimension_semantics=("parallel", "parallel"),
        ),
    )(xyzp8, xyzt8)


def _sc_gather(featf, xyzw, idxf):
    r_total = idxf.shape[0] * idxf.shape[1]
    c = featf.shape[1]
    npts = xyzw.shape[0] // 4
    rpw = r_total // NW
    nch = rpw // CH
    mesh = plsc.VectorSubcoreMesh(core_axis_name="c", subcore_axis_name="s")

    @functools.partial(
        pl.kernel,
        mesh=mesh,
        out_type=(
            jax.ShapeDtypeStruct((r_total, c), jnp.float32),
            jax.ShapeDtypeStruct((r_total * 3,), jnp.float32),
        ),
        scratch_types=[
            pltpu.VMEM((rpw // 128, 128), jnp.int32),
            pltpu.VMEM((2, CH, c), jnp.float32),
            pltpu.VMEM((npts * 4,), jnp.float32),
            pltpu.VMEM((2, CH * 3), jnp.float32),
            [pltpu.SemaphoreType.DMA] * 2,
            [pltpu.SemaphoreType.DMA] * 2,
        ],
        compiler_params=pltpu.CompilerParams(needs_layout_passes=False),
    )
    def gather(feat_hbm, xyzw_hbm, idx_hbm, featg_hbm, xyzg_hbm,
               idx_v, featr_v, xyztab_v, xyzo_v, gsem, wsem):
        wid = lax.axis_index("s") * NC + lax.axis_index("c")
        base0 = wid * rpw
        pltpu.sync_copy(idx_hbm.at[pl.ds(wid * (rpw // 128), rpw // 128)],
                        idx_v)
        pltpu.sync_copy(xyzw_hbm, xyztab_v)
        iota16 = lax.iota(jnp.int32, 16)
        qg0 = wid * (rpw // KNB)

        def start_gathers(ci, p):
            for j in range(CH // 128):
                pltpu.async_copy(
                    feat_hbm.at[idx_v.at[ci * (CH // 128) + j]],
                    featr_v.at[p].at[pl.ds(j * 128, 128)], gsem[p])

        def finish(ci, p):
            for v in range(CH * 3 // 16):
                wl = ci * (CH * 3) + v * 16 + iota16
                r = wl // 3
                col = wl - r * 3
                q = qg0 + (r >> 5)
                nidx = plsc.load_gather(idx_v, [r >> 7, r & 127])
                xv = plsc.load_gather(xyztab_v, [nidx * 4 + col])
                cv = plsc.load_gather(xyztab_v, [q * 4 + col])
                xyzo_v[p, pl.ds(v * 16, 16)] = xv - cv
            base = base0 + ci * CH
            for j in range(CH // 128):
                pltpu.make_async_copy(
                    feat_hbm.at[idx_v.at[ci * (CH // 128) + j]],
                    featr_v.at[p].at[pl.ds(j * 128, 128)], gsem[p]).wait()
            pltpu.async_copy(featr_v.at[p], featg_hbm.at[pl.ds(base, CH)],
                             wsem[p])
            pltpu.async_copy(xyzo_v.at[p],
                             xyzg_hbm.at[pl.ds(base * 3, CH * 3)], wsem[p])

        def wait_writeout(ci, p):
            base = base0 + ci * CH
            pltpu.make_async_copy(featr_v.at[p],
                                  featg_hbm.at[pl.ds(base, CH)],
                                  wsem[p]).wait()
            pltpu.make_async_copy(xyzo_v.at[p],
                                  xyzg_hbm.at[pl.ds(base * 3, CH * 3)],
                                  wsem[p]).wait()

        start_gathers(0, 0)

        def pair(j, carry):
            c0 = j * 2
            @pl.when(j > 0)
            def _():
                wait_writeout(c0 - 1, 1)
            start_gathers(c0 + 1, 1)
            finish(c0, 0)
            wait_writeout(c0, 0)
            @pl.when(c0 + 2 < nch)
            def _():
                start_gathers(c0 + 2, 0)
            finish(c0 + 1, 1)
            return carry

        lax.fori_loop(0, nch // 2, pair, 0)
        wait_writeout(nch - 1, 1)

    return gather(featf, xyzw, idxf)


def kernel(xyz, feat):
    b, n, _ = xyz.shape
    c = feat.shape[-1]
    xyzp8 = jnp.pad(xyz, ((0, 0), (0, 0), (0, 5)))
    xyzt8 = jnp.transpose(xyzp8, (0, 2, 1))
    idx = _tc_knn(xyzp8, xyzt8)
    featf = feat.reshape(b * n, c)
    xyzw = jnp.pad(xyz.reshape(b * n, 3), ((0, 0), (0, 1))).reshape(-1)
    featg, xyzg = _sc_gather(featf, xyzw, idx.reshape(-1, 128))
    neighborhood = xyzg.reshape(b, n, KNB, 3)
    neighborhood_feat = featg.reshape(b, n, KNB, c)
    return (neighborhood, neighborhood_feat)


# speedup vs baseline: 15.1233x; 1.0770x over previous
import functools

import jax
import jax.numpy as jnp
from jax import lax
from jax.experimental import pallas as pl
from jax.experimental.pallas import tpu as pltpu
from jax.experimental.pallas import tpu_sc as plsc

KNB = 32
QB = 256
NC = 2
NS = 16
NW = NC * NS
CH = 256
XW = 16


def _knn_body(xyzq_ref, xyzt_ref, idx_ref):
    n = xyzt_ref.shape[-1]
    lhs = xyzq_ref[...]
    rhs = xyzt_ref[...]
    qx, qy, qz = lhs[:, 0:1], lhs[:, 1:2], lhs[:, 2:3]
    q2 = (qx * qx + qy * qy) + qz * qz
    rx, ry, rz = rhs[0:1, :], rhs[1:2, :], rhs[2:3, :]
    r2 = (rx * rx + ry * ry) + rz * rz
    inner = jnp.dot(lhs, rhs, preferred_element_type=jnp.float32)
    d2 = (q2 + r2) - 2.0 * inner
    lanes = lax.broadcasted_iota(jnp.int32, d2.shape, 1).astype(jnp.float32)
    nf = jnp.float32(n)
    cols = []
    for _ in range(KNB):
        m = jnp.min(d2, axis=1, keepdims=True)
        am = jnp.min(jnp.where(d2 == m, lanes, nf), axis=1, keepdims=True)
        cols.append(am)
        d2 = jnp.where(lanes == am, jnp.float32(jnp.inf), d2)
    idx_ref[...] = jnp.concatenate(cols, axis=1).astype(jnp.int32)


def _tc_knn(xyzp8b, xyzt8b):
    n = xyzp8b.shape[0]
    return pl.pallas_call(
        _knn_body,
        grid=(n // QB,),
        in_specs=[
            pl.BlockSpec((QB, 8), lambda j: (j, 0)),
            pl.BlockSpec((8, n), lambda j: (0, 0)),
        ],
        out_specs=pl.BlockSpec((QB, KNB), lambda j: (j, 0)),
        out_shape=jax.ShapeDtypeStruct((n, KNB), jnp.int32),
        compiler_params=pltpu.CompilerParams(
            dimension_semantics=("parallel",),
        ),
    )(xyzp8b, xyzt8b)


def _sc_gather(bi, featf, xyzw, idxf, featg_ref, xyzg_ref):
    r_total = idxf.shape[0] * idxf.shape[1]
    c = featf.shape[1]
    npts = xyzw.shape[0] // 4
    rpw = r_total // NW
    nch = rpw // CH
    mesh = plsc.VectorSubcoreMesh(core_axis_name="c", subcore_axis_name="s")

    @functools.partial(
        pl.kernel,
        mesh=mesh,
        scratch_types=[
            pltpu.VMEM((rpw // 128, 128), jnp.int32),
            pltpu.VMEM((2, CH, c), jnp.float32),
            pltpu.VMEM((npts * 4,), jnp.float32),
            pltpu.VMEM((2, CH * 3), jnp.float32),
            [pltpu.SemaphoreType.DMA] * 2,
            [pltpu.SemaphoreType.DMA] * 2,
        ],
        compiler_params=pltpu.CompilerParams(needs_layout_passes=False),
    )
    def gather(feat_hbm, xyzw_hbm, idx_hbm, featg_hbm, xyzg_hbm,
               idx_v, featr_v, xyztab_v, xyzo_v, gsem, wsem):
        wid = lax.axis_index("s") * NC + lax.axis_index("c")
        base0 = bi * r_total + wid * rpw
        pltpu.sync_copy(idx_hbm.at[pl.ds(wid * (rpw // 128), rpw // 128)],
                        idx_v)
        pltpu.sync_copy(xyzw_hbm, xyztab_v)
        iota16 = lax.iota(jnp.int32, 16)
        qg0 = wid * (rpw // KNB)

        def start_gathers(ci, p):
            for j in range(CH // 128):
                pltpu.async_copy(
                    feat_hbm.at[idx_v.at[ci * (CH // 128) + j]],
                    featr_v.at[p].at[pl.ds(j * 128, 128)], gsem[p])

        def finish(ci, p):
            for v in range(CH * 3 // 16):
                wl = ci * (CH * 3) + v * 16 + iota16
                r = wl // 3
                col = wl - r * 3
                q = qg0 + (r >> 5)
                nidx = plsc.load_gather(idx_v, [r >> 7, r & 127])
                xv = plsc.load_gather(xyztab_v, [nidx * 4 + col])
                cv = plsc.load_gather(xyztab_v, [q * 4 + col])
                xyzo_v[p, pl.ds(v * 16, 16)] = xv - cv
            base = base0 + ci * CH
            for j in range(CH // 128):
                pltpu.make_async_copy(
                    feat_hbm.at[idx_v.at[ci * (CH // 128) + j]],
                    featr_v.at[p].at[pl.ds(j * 128, 128)], gsem[p]).wait()
            pltpu.async_copy(featr_v.at[p], featg_hbm.at[pl.ds(base, CH)],
                             wsem[p])
            pltpu.async_copy(xyzo_v.at[p],
                             xyzg_hbm.at[pl.ds(base * 3, CH * 3)], wsem[p])

        def wait_writeout(ci, p):
            base = base0 + ci * CH
            pltpu.make_async_copy(featr_v.at[p],
                                  featg_hbm.at[pl.ds(base, CH)],
                                  wsem[p]).wait()
            pltpu.make_async_copy(xyzo_v.at[p],
                                  xyzg_hbm.at[pl.ds(base * 3, CH * 3)],
                                  wsem[p]).wait()

        start_gathers(0, 0)

        def pair(j, carry):
            c0 = j * 2
            @pl.when(j > 0)
            def _():
                wait_writeout(c0 - 1, 1)
            start_gathers(c0 + 1, 1)
            finish(c0, 0)
            wait_writeout(c0, 0)
            @pl.when(c0 + 2 < nch)
            def _():
                start_gathers(c0 + 2, 0)
            finish(c0 + 1, 1)
            return carry

        lax.fori_loop(0, nch // 2, pair, 0)
        wait_writeout(nch - 1, 1)

    gather(featf, xyzw, idxf, featg_ref, xyzg_ref)


def kernel(xyz, feat):
    b, n, _ = xyz.shape
    c = feat.shape[-1]
    r_b = n * KNB
    featg_ref = jax.new_ref(pl.empty((b * r_b, c), jnp.float32))
    xyzg_ref = jax.new_ref(pl.empty((b * r_b * 3,), jnp.float32))
    xyzp8 = jnp.pad(xyz, ((0, 0), (0, 0), (0, 5)))
    xyzt8 = jnp.transpose(xyzp8, (0, 2, 1))
    for bi in range(b):
        idx_b = _tc_knn(xyzp8[bi], xyzt8[bi])
        xyzw_b = jnp.pad(xyz[bi], ((0, 0), (0, 1))).reshape(-1)
        _sc_gather(bi, feat[bi], xyzw_b, idx_b.reshape(-1, 128),
                   featg_ref, xyzg_ref)
    neighborhood = jax.freeze(xyzg_ref).reshape(b, n, KNB, 3)
    neighborhood_feat = jax.freeze(featg_ref).reshape(b, n, KNB, c)
    return (neighborhood, neighborhood_feat)


# speedup vs baseline: 20.4260x; 1.3506x over previous
import functools

import jax
import jax.numpy as jnp
from jax import lax
from jax.experimental import pallas as pl
from jax.experimental.pallas import tpu as pltpu
from jax.experimental.pallas import tpu_sc as plsc

KNB = 32
QB = 256
NC = 2
NS = 16
NW = NC * NS
CH = 256
XW = 16


def _knn_body(xyzq_ref, xyzt_ref, idx_ref):
    n = xyzt_ref.shape[-1]
    lhs = xyzq_ref[...]
    rhs = xyzt_ref[...]
    qx, qy, qz = lhs[:, 0:1], lhs[:, 1:2], lhs[:, 2:3]
    q2 = (qx * qx + qy * qy) + qz * qz
    rx, ry, rz = rhs[0:1, :], rhs[1:2, :], rhs[2:3, :]
    r2 = (rx * rx + ry * ry) + rz * rz
    inner = jnp.dot(lhs, rhs, preferred_element_type=jnp.float32)
    d2 = (q2 + r2) - 2.0 * inner
    lanes = lax.broadcasted_iota(jnp.int32, d2.shape, 1).astype(jnp.float32)
    nf = jnp.float32(n)
    cols = []
    for _ in range(KNB):
        m = jnp.min(d2, axis=1, keepdims=True)
        am = jnp.min(jnp.where(d2 == m, lanes, nf), axis=1, keepdims=True)
        cols.append(am)
        d2 = jnp.where(lanes == am, jnp.float32(jnp.inf), d2)
    idx_ref[...] = jnp.concatenate(cols, axis=1).astype(jnp.int32)


def _tc_knn(xyzp8b, xyzt8b):
    n = xyzp8b.shape[0]
    return pl.pallas_call(
        _knn_body,
        grid=(n // QB,),
        in_specs=[
            pl.BlockSpec((QB, 8), lambda j: (j, 0)),
            pl.BlockSpec((8, n), lambda j: (0, 0)),
        ],
        out_specs=pl.BlockSpec((QB, KNB), lambda j: (j, 0)),
        out_shape=jax.ShapeDtypeStruct((n, KNB), jnp.int32),
        compiler_params=pltpu.CompilerParams(
            dimension_semantics=("parallel",),
        ),
    )(xyzp8b, xyzt8b)


def _sc_gather(bi, featf, xyzw, idxf, featg_ref, xyzg_ref):
    r_total = idxf.shape[0] * idxf.shape[1]
    c = featf.shape[1]
    npts = xyzw.shape[0] // 4
    rpw = r_total // NW
    chf = 128
    nch = rpw // chf
    nplanes = 3 * (KNB // 8)
    mesh = plsc.VectorSubcoreMesh(core_axis_name="c", subcore_axis_name="s")

    @functools.partial(
        pl.kernel,
        mesh=mesh,
        scratch_types=[
            pltpu.VMEM((r_total // 128, 128), jnp.int32),
            pltpu.VMEM((2, chf, c), jnp.float32),
            pltpu.VMEM((npts * 4,), jnp.float32),
            pltpu.VMEM((8, npts), jnp.float32),
            [pltpu.SemaphoreType.DMA] * 2,
            [pltpu.SemaphoreType.DMA] * 2,
            pltpu.SemaphoreType.DMA,
        ],
        compiler_params=pltpu.CompilerParams(needs_layout_passes=False),
    )
    def gather(feat_hbm, xyzw_hbm, idx_hbm, featg_hbm, xyzg_hbm,
               idx_v, featr_v, xyztab_v, xyzpl_v, gsem, wsem, psem):
        wid = lax.axis_index("s") * NC + lax.axis_index("c")
        base0 = bi * r_total + wid * rpw
        pltpu.sync_copy(idx_hbm, idx_v)
        pltpu.sync_copy(xyzw_hbm, xyztab_v)
        iota16 = lax.iota(jnp.int32, 16)

        def start_gathers(ci, p):
            pltpu.async_copy(feat_hbm.at[idx_v.at[wid * nch + ci]],
                             featr_v.at[p], gsem[p])

        def finish(ci, p):
            base = base0 + ci * chf
            pltpu.make_async_copy(feat_hbm.at[idx_v.at[wid * nch + ci]],
                                  featr_v.at[p], gsem[p]).wait()
            pltpu.async_copy(featr_v.at[p], featg_hbm.at[pl.ds(base, chf)],
                             wsem[p])

        def wait_writeout(ci, p):
            base = base0 + ci * chf
            pltpu.make_async_copy(featr_v.at[p],
                                  featg_hbm.at[pl.ds(base, chf)],
                                  wsem[p]).wait()

        start_gathers(0, 0)

        @pl.when(wid < nplanes)
        def _():
            cc = wid // (KNB // 8)
            kb = wid % (KNB // 8)
            for kk in range(8):
                k = kb * 8 + kk

                def vloop(v, carry):
                    nq = v * 16 + iota16
                    r = nq * KNB + k
                    nidx = plsc.load_gather(idx_v, [r >> 7, r & 127])
                    xv = plsc.load_gather(xyztab_v, [nidx * 4 + cc])
                    cv = plsc.load_gather(xyztab_v, [nq * 4 + cc])
                    xyzpl_v[kk, pl.ds(v * 16, 16)] = xv - cv
                    return carry

                lax.fori_loop(0, npts // 16, vloop, 0)
            pltpu.async_copy(
                xyzpl_v,
                xyzg_hbm.at[bi].at[cc].at[pl.ds(kb * 8, 8)], psem)

        def pair(j, carry):
            c0 = j * 2
            @pl.when(j > 0)
            def _():
                wait_writeout(c0 - 1, 1)
            start_gathers(c0 + 1, 1)
            finish(c0, 0)
            wait_writeout(c0, 0)
            @pl.when(c0 + 2 < nch)
            def _():
                start_gathers(c0 + 2, 0)
            finish(c0 + 1, 1)
            return carry

        lax.fori_loop(0, nch // 2, pair, 0)
        wait_writeout(nch - 1, 1)

        @pl.when(wid < nplanes)
        def _():
            pltpu.make_async_copy(
                xyzpl_v,
                xyzg_hbm.at[bi].at[wid // (KNB // 8)]
                .at[pl.ds((wid % (KNB // 8)) * 8, 8)], psem).wait()

    gather(featf, xyzw, idxf, featg_ref, xyzg_ref)


def kernel(xyz, feat):
    b, n, _ = xyz.shape
    c = feat.shape[-1]
    r_b = n * KNB
    featg_ref = jax.new_ref(pl.empty((b * r_b, c), jnp.float32))
    xyzg_ref = jax.new_ref(pl.empty((b, 3, KNB, n), jnp.float32))
    xyzp8 = jnp.pad(xyz, ((0, 0), (0, 0), (0, 5)))
    xyzt8 = jnp.transpose(xyzp8, (0, 2, 1))
    for bi in range(b):
        idx_b = _tc_knn(xyzp8[bi], xyzt8[bi])
        xyzw_b = jnp.pad(xyz[bi], ((0, 0), (0, 1))).reshape(-1)
        _sc_gather(bi, feat[bi], xyzw_b, idx_b.reshape(-1, 128),
                   featg_ref, xyzg_ref)
    neighborhood = jnp.transpose(jax.freeze(xyzg_ref), (0, 3, 2, 1))
    neighborhood_feat = jax.freeze(featg_ref).reshape(b, n, KNB, c)
    return (neighborhood, neighborhood_feat)
